# Initial kernel scaffold; baseline (speedup 1.0000x reference)
import jax
import jax.numpy as jnp
from jax.experimental import pallas as pl


def kernel(x, edge_index, W1, a_src1, a_dst1, b1, W2, a_src2, a_dst2, b2):
    raise NotImplementedError("write your pallas kernel here")


# speedup vs baseline: 1.1424x; 1.1424x over previous
import jax
import jax.numpy as jnp
from jax.experimental import pallas as pl


def _proj_kernel(x_ref, w_ref, asr_ref, adr_ref, h_ref, as_ref, ad_ref, mx_ref):
    h = jnp.dot(x_ref[...], w_ref[...], preferred_element_type=jnp.float32,
                precision=jax.lax.Precision.HIGHEST)
    h_ref[...] = h
    a_s = jnp.sum(h * asr_ref[...], axis=1)
    a_d = jnp.sum(h * adr_ref[...], axis=1)
    as_ref[...] = a_s
    ad_ref[...] = a_d
    mx_ref[...] = jnp.full((16,), jnp.max(a_s), dtype=jnp.float32)


def _project(x, W, a_src, a_dst):
    N = x.shape[0]
    return pl.pallas_call(
        _proj_kernel,
        out_shape=(
            jax.ShapeDtypeStruct((N, W.shape[1]), jnp.float32),
            jax.ShapeDtypeStruct((N,), jnp.float32),
            jax.ShapeDtypeStruct((N,), jnp.float32),
            jax.ShapeDtypeStruct((16,), jnp.float32),
        ),
    )(x, W, a_src[None, :], a_dst[None, :])


def _leaky_relu(x, slope=0.2):
    return jnp.where(x >= 0, x, slope * x)


def _gat_layer(x, edge_index, W, a_src, a_dst, b):
    src = edge_index[0]
    dst = edge_index[1]
    N = x.shape[0]
    h, alpha_src, alpha_dst, _ = _project(x, W, a_src, a_dst)
    e = _leaky_relu(alpha_src[src] + alpha_dst[dst], 0.2)
    m = jax.ops.segment_max(e, dst, num_segments=N)
    m = jnp.where(jnp.isfinite(m), m, 0.0)
    ex = jnp.exp(e - m[dst])
    s = jax.ops.segment_sum(ex, dst, num_segments=N)
    alpha = ex / (s[dst] + 1e-16)
    out = jax.ops.segment_sum(alpha[:, None] * h[src], dst, num_segments=N)
    return out + b


def kernel(x, edge_index, W1, a_src1, a_dst1, b1, W2, a_src2, a_dst2, b2):
    h = _gat_layer(x, edge_index, W1, a_src1, a_dst1, b1)
    h = jax.nn.relu(h)
    out = _gat_layer(h, edge_index, W2, a_src2, a_dst2, b2)
    return out
